# speedup vs baseline: 2.2880x; 2.2880x over previous
import functools

import jax
import jax.numpy as jnp
from jax import lax
from jax.experimental import pallas as pl
from jax.experimental.pallas import tpu as pltpu

N_NODES = 10000
N_EDGES = 320000
D = 128
D5 = 5 * D
ATTR = 33

EDGE_BLOCK = 1280


def _edge_block_kernel(attr_ref, xs_ref, xd_ref, w1p_ref, b_ref, wt_ref,
                       md_ref, ms_ref):
    attr = attr_ref[...]
    w = jax.nn.silu(
        jnp.dot(attr, w1p_ref[...], preferred_element_type=jnp.float32)
        + b_ref[...])
    xs = xs_ref[...]
    xd = xd_ref[...]
    g = w * (xs - xd)
    a = 0.5 * w * (xs + xd)
    dxe = jnp.concatenate([g, a, g * a, g * g, a * a], axis=1)
    x = jnp.tanh(dxe)
    x = jnp.dot(x, wt_ref[...], preferred_element_type=jnp.float32)
    x = x - jnp.mean(x, axis=1, keepdims=True)
    x = x * lax.rsqrt(jnp.sum(x * x, axis=1, keepdims=True) + 0.001)
    x = jnp.tanh(x)
    x = jnp.dot(x, wt_ref[...], preferred_element_type=jnp.float32)
    dxe2 = jnp.tanh(x)
    g2 = w * dxe2[:, :D]
    s2 = 0.5 * w * (dxe2[:, D:2 * D] + dxe2[:, 2 * D:3 * D]
                    + dxe2[:, 3 * D:4 * D] + dxe2[:, 4 * D:])
    md_ref[...] = g2 + s2
    ms_ref[...] = s2 - g2


def _edge_mlp(xe_attr, xs, xd, fc1_w, fc1_b, dl_w1, *, interpret=False):
    nb = N_EDGES // EDGE_BLOCK
    w1p = fc1_w.T
    b = fc1_b.reshape(1, D)
    wt = dl_w1.T
    md, ms = pl.pallas_call(
        _edge_block_kernel,
        grid=(nb,),
        in_specs=[
            pl.BlockSpec((EDGE_BLOCK, ATTR), lambda i: (i, 0)),
            pl.BlockSpec((EDGE_BLOCK, D), lambda i: (i, 0)),
            pl.BlockSpec((EDGE_BLOCK, D), lambda i: (i, 0)),
            pl.BlockSpec((ATTR, D), lambda i: (0, 0)),
            pl.BlockSpec((1, D), lambda i: (0, 0)),
            pl.BlockSpec((D5, D5), lambda i: (0, 0)),
        ],
        out_specs=[
            pl.BlockSpec((EDGE_BLOCK, D), lambda i: (i, 0)),
            pl.BlockSpec((EDGE_BLOCK, D), lambda i: (i, 0)),
        ],
        out_shape=[
            jax.ShapeDtypeStruct((N_EDGES, D), jnp.float32),
            jax.ShapeDtypeStruct((N_EDGES, D), jnp.float32),
        ],
        interpret=interpret,
    )(xe_attr, xs, xd, w1p, b, wt)
    return md, ms


def kernel(xn, xe_attr, xe_src, xe_dst, fc1_w, fc1_b, dl_w1, dl_w2):
    del dl_w2
    xs = jnp.take(xn, xe_src, axis=0)
    xd = jnp.take(xn, xe_dst, axis=0)
    md, ms = _edge_mlp(xe_attr, xs, xd, fc1_w, fc1_b, dl_w1)
    out = (jax.ops.segment_sum(md, xe_dst, num_segments=N_NODES)
           + jax.ops.segment_sum(ms, xe_src, num_segments=N_NODES))
    return out


# speedup vs baseline: 4.7973x; 2.0967x over previous
import functools

import jax
import jax.numpy as jnp
from jax import lax
from jax.experimental import pallas as pl
from jax.experimental.pallas import tpu as pltpu
from jax.experimental.pallas import tpu_sc as plsc

N_NODES = 10000
N_EDGES = 320000
D = 128
D5 = 5 * D
ATTR = 33

EDGE_BLOCK = 1280

NC = 2
NS = 16
NW = NC * NS
EPW = N_EDGES // NW
CH = 80
NCHUNK = EPW // CH
NP = 10240

_SC_MESH = dict(core_axis_name="c", subcore_axis_name="s")


def _gather_body(xn_hbm, src_hbm, dst_hbm, xs_hbm, xd_hbm,
                 idx_v, rows_v, sem):
    wid = lax.axis_index("s") * NC + lax.axis_index("c")

    def body(i, carry):
        off = wid * EPW + i * CH
        pltpu.sync_copy(src_hbm.at[pl.ds(off, CH)], idx_v)
        pltpu.async_copy(xn_hbm.at[idx_v], rows_v, sem).wait()
        pltpu.sync_copy(rows_v, xs_hbm.at[pl.ds(off, CH)])
        pltpu.sync_copy(dst_hbm.at[pl.ds(off, CH)], idx_v)
        pltpu.async_copy(xn_hbm.at[idx_v], rows_v, sem).wait()
        pltpu.sync_copy(rows_v, xd_hbm.at[pl.ds(off, CH)])
        return carry

    lax.fori_loop(0, NCHUNK, body, 0)


def _sc_gather(xn, xe_src, xe_dst):
    return pl.kernel(
        _gather_body,
        out_type=[
            jax.ShapeDtypeStruct((N_EDGES, D), jnp.float32),
            jax.ShapeDtypeStruct((N_EDGES, D), jnp.float32),
        ],
        mesh=plsc.VectorSubcoreMesh(**_SC_MESH),
        scratch_types=[
            pltpu.VMEM((CH,), jnp.int32),
            pltpu.VMEM((CH, D), jnp.float32),
            pltpu.SemaphoreType.DMA,
        ],
    )(xn, xe_src, xe_dst)


def _scatter_body(md_hbm, ms_hbm, dst_hbm, src_hbm, zeros_hbm, out_hbm,
                  idx_v, rows_v, acc_sh):
    cid = lax.axis_index("c")
    sid = lax.axis_index("s")
    wid = sid * NC + cid
    zr = NP // NS
    pltpu.sync_copy(zeros_hbm.at[pl.ds(sid * zr, zr)],
                    acc_sh.at[pl.ds(sid * zr, zr)])
    plsc.subcore_barrier()

    def body(i, carry):
        off = wid * EPW + i * CH
        pltpu.sync_copy(dst_hbm.at[pl.ds(off, CH)], idx_v)
        pltpu.sync_copy(md_hbm.at[pl.ds(off, CH)], rows_v)
        pltpu.sync_copy(rows_v, acc_sh.at[idx_v], add=True)
        pltpu.sync_copy(src_hbm.at[pl.ds(off, CH)], idx_v)
        pltpu.sync_copy(ms_hbm.at[pl.ds(off, CH)], rows_v)
        pltpu.sync_copy(rows_v, acc_sh.at[idx_v], add=True)
        return carry

    lax.fori_loop(0, NCHUNK, body, 0)
    plsc.subcore_barrier()
    pltpu.sync_copy(acc_sh.at[pl.ds(sid * zr, zr)],
                    out_hbm.at[cid].at[pl.ds(sid * zr, zr)])


def _sc_scatter(md, ms, xe_dst, xe_src):
    zeros = jnp.zeros((NP, D), jnp.float32)
    return pl.kernel(
        _scatter_body,
        out_type=jax.ShapeDtypeStruct((NC, NP, D), jnp.float32),
        mesh=plsc.VectorSubcoreMesh(**_SC_MESH),
        scratch_types=[
            pltpu.VMEM((CH,), jnp.int32),
            pltpu.VMEM((CH, D), jnp.float32),
            pltpu.VMEM_SHARED((NP, D), jnp.float32),
        ],
    )(md, ms, xe_dst, xe_src, zeros)


def _edge_block_kernel(attr_ref, xs_ref, xd_ref, w1p_ref, b_ref, wt_ref,
                       md_ref, ms_ref):
    attr = attr_ref[...]
    w = jax.nn.silu(
        jnp.dot(attr, w1p_ref[...], preferred_element_type=jnp.float32)
        + b_ref[...])
    xs = xs_ref[...]
    xd = xd_ref[...]
    g = w * (xs - xd)
    a = 0.5 * w * (xs + xd)
    dxe = jnp.concatenate([g, a, g * a, g * g, a * a], axis=1)
    x = jnp.tanh(dxe)
    x = jnp.dot(x, wt_ref[...], preferred_element_type=jnp.float32)
    x = x - jnp.mean(x, axis=1, keepdims=True)
    x = x * lax.rsqrt(jnp.sum(x * x, axis=1, keepdims=True) + 0.001)
    x = jnp.tanh(x)
    x = jnp.dot(x, wt_ref[...], preferred_element_type=jnp.float32)
    dxe2 = jnp.tanh(x)
    g2 = w * dxe2[:, :D]
    s2 = 0.5 * w * (dxe2[:, D:2 * D] + dxe2[:, 2 * D:3 * D]
                    + dxe2[:, 3 * D:4 * D] + dxe2[:, 4 * D:])
    md_ref[...] = g2 + s2
    ms_ref[...] = s2 - g2


def _edge_mlp(xe_attr, xs, xd, fc1_w, fc1_b, dl_w1, *, interpret=False):
    nb = N_EDGES // EDGE_BLOCK
    w1p = fc1_w.T
    b = fc1_b.reshape(1, D)
    wt = dl_w1.T
    md, ms = pl.pallas_call(
        _edge_block_kernel,
        grid=(nb,),
        in_specs=[
            pl.BlockSpec((EDGE_BLOCK, ATTR), lambda i: (i, 0)),
            pl.BlockSpec((EDGE_BLOCK, D), lambda i: (i, 0)),
            pl.BlockSpec((EDGE_BLOCK, D), lambda i: (i, 0)),
            pl.BlockSpec((ATTR, D), lambda i: (0, 0)),
            pl.BlockSpec((1, D), lambda i: (0, 0)),
            pl.BlockSpec((D5, D5), lambda i: (0, 0)),
        ],
        out_specs=[
            pl.BlockSpec((EDGE_BLOCK, D), lambda i: (i, 0)),
            pl.BlockSpec((EDGE_BLOCK, D), lambda i: (i, 0)),
        ],
        out_shape=[
            jax.ShapeDtypeStruct((N_EDGES, D), jnp.float32),
            jax.ShapeDtypeStruct((N_EDGES, D), jnp.float32),
        ],
        interpret=interpret,
    )(xe_attr, xs, xd, w1p, b, wt)
    return md, ms


def kernel(xn, xe_attr, xe_src, xe_dst, fc1_w, fc1_b, dl_w1, dl_w2):
    del dl_w2
    xs, xd = _sc_gather(xn, xe_src, xe_dst)
    md, ms = _edge_mlp(xe_attr, xs, xd, fc1_w, fc1_b, dl_w1)
    partials = _sc_scatter(md, ms, xe_dst, xe_src)
    return (partials[0] + partials[1])[:N_NODES]
